# baseline (speedup 1.0000x reference)
import jax
import jax.numpy as jnp
from jax.experimental import pallas as pl
from jax.experimental.pallas import tpu as pltpu

_N = 4096
_C = 256
_EPS = 1e-5
_B = 256
_G = _N // _B
_T = 2048
_NT = _N // _T


def _bn_coeffs(v, gamma, beta):
    mean = jnp.mean(v, axis=0, keepdims=True)
    var = jnp.mean((v - mean) ** 2, axis=0, keepdims=True)
    rstd = gamma / jnp.sqrt(var + _EPS)
    return rstd, beta - mean * rstd


def _body(lap_ref, x_ref, w1_ref, b1_ref, g1_ref, be1_ref,
          w2_ref, b2_ref, g2_ref, be2_ref,
          out_ref, lbf, v0, v1, v2, hf, c1):
    g = pl.program_id(0)

    @pl.when(g == 0)
    def _():
        s1, o1 = _bn_coeffs(x_ref[...], g1_ref[...], be1_ref[...])
        c1[0:1, :] = s1
        c1[1:2, :] = o1
        v0[...] = (x_ref[...] * s1 + o1).astype(jnp.bfloat16)

    blk = lap_ref[...].astype(jnp.bfloat16)
    lbf[pl.ds(g * _B, _B), :] = blk
    v1[pl.ds(g * _B, _B), :] = jnp.dot(
        blk, v0[...], preferred_element_type=jnp.float32
    ).astype(jnp.bfloat16)

    @pl.when(g == _G - 1)
    def _():
        w1 = w1_ref[...]
        w2 = w2_ref[...]

        def mmb(a, b):
            return jnp.dot(a, b, preferred_element_type=jnp.float32).astype(
                jnp.bfloat16
            )

        def mmf(a, b):
            return jnp.dot(a, b, preferred_element_type=jnp.float32)

        def rows(ref, i):
            return ref[pl.ds(i * _T, _T), :]

        def sl(i):
            return (pl.ds(i * _T, _T), slice(None))

        for i in range(_NT):
            v2[sl(i)] = 2.0 * mmb(rows(lbf, i), v1[...]) - rows(v0, i)
        hsum = []
        hsq = []
        for i in range(_NT):
            t3_t = 2.0 * mmb(rows(lbf, i), v2[...]) - rows(v1, i)
            acc_t = (mmf(rows(v0, i), w1[0]) + mmf(rows(v1, i), w1[1])) + (
                mmf(rows(v2, i), w1[2]) + mmf(t3_t, w1[3])
            )
            h_t = jnp.maximum(acc_t + b1_ref[...], 0.0)
            hf[sl(i)] = h_t
            hsum.append(jnp.sum(h_t, axis=0, keepdims=True))
            hsq.append(jnp.sum(h_t * h_t, axis=0, keepdims=True))
        mean2 = sum(hsum) * (1.0 / _N)
        var2 = sum(hsq) * (1.0 / _N) - mean2 * mean2
        s2 = g2_ref[...] / jnp.sqrt(var2 + _EPS)
        o2 = be2_ref[...] - mean2 * s2
        for i in range(_NT):
            v0[sl(i)] = (rows(hf, i) * s2 + o2).astype(jnp.bfloat16)
        for i in range(_NT):
            v1[sl(i)] = mmb(rows(lbf, i), v0[...])
        for i in range(_NT):
            v2[sl(i)] = 2.0 * mmb(rows(lbf, i), v1[...]) - rows(v0, i)
        s1 = c1[0:1, :]
        o1 = c1[1:2, :]
        for i in range(_NT):
            u3_t = 2.0 * mmb(rows(lbf, i), v2[...]) - rows(v1, i)
            acc_t = (mmf(rows(v0, i), w2[0]) + mmf(rows(v1, i), w2[1])) + (
                mmf(rows(v2, i), w2[2]) + mmf(u3_t, w2[3])
            )
            xh_t = rows(x_ref, i) * s1 + o1
            out_ref[sl(i)] = jnp.maximum(xh_t + acc_t + b2_ref[...], 0.0)


def kernel(x, laplacian, bn1_gamma, bn1_beta, W1, b1, bn2_gamma, bn2_beta, W2, b2):
    w1b = W1.astype(jnp.bfloat16)
    w2b = W2.astype(jnp.bfloat16)
    b1r = b1.reshape(1, _C)
    b2r = b2.reshape(1, _C)
    g1r = bn1_gamma.reshape(1, _C)
    be1r = bn1_beta.reshape(1, _C)
    g2r = bn2_gamma.reshape(1, _C)
    be2r = bn2_beta.reshape(1, _C)

    full = lambda shape: pl.BlockSpec(shape, lambda g: tuple(0 for _ in shape))
    return pl.pallas_call(
        _body,
        grid=(_G,),
        in_specs=[
            pl.BlockSpec((_B, _N), lambda g: (g, 0)),
            full((_N, _C)),
            full((4, _C, _C)),
            full((1, _C)),
            full((1, _C)),
            full((1, _C)),
            full((4, _C, _C)),
            full((1, _C)),
            full((1, _C)),
            full((1, _C)),
        ],
        out_specs=full((_N, _C)),
        out_shape=jax.ShapeDtypeStruct((_N, _C), jnp.float32),
        scratch_shapes=[
            pltpu.VMEM((_N, _N), jnp.bfloat16),
            pltpu.VMEM((_N, _C), jnp.bfloat16),
            pltpu.VMEM((_N, _C), jnp.bfloat16),
            pltpu.VMEM((_N, _C), jnp.bfloat16),
            pltpu.VMEM((_N, _C), jnp.float32),
            pltpu.VMEM((2, _C), jnp.float32),
        ],
        compiler_params=pltpu.CompilerParams(
            dimension_semantics=("arbitrary",),
            vmem_limit_bytes=64 * 1024 * 1024,
        ),
    )(laplacian, x, w1b, b1r, g1r, be1r, w2b, b2r, g2r, be2r)


# speedup vs baseline: 1.0178x; 1.0178x over previous
import jax
import jax.numpy as jnp
from jax.experimental import pallas as pl
from jax.experimental.pallas import tpu as pltpu

_N = 4096
_C = 256
_EPS = 1e-5
_B = 256
_G = _N // _B
_T = 1024
_NT = _N // _T


def _bn_coeffs(v, gamma, beta):
    mean = jnp.mean(v, axis=0, keepdims=True)
    var = jnp.mean((v - mean) ** 2, axis=0, keepdims=True)
    rstd = gamma / jnp.sqrt(var + _EPS)
    return rstd, beta - mean * rstd


def _body(lap_ref, x_ref, w1_ref, b1_ref, g1_ref, be1_ref,
          w2_ref, b2_ref, g2_ref, be2_ref,
          out_ref, lbf, v0, v1, v2, hf, c1):
    g = pl.program_id(0)

    @pl.when(g == 0)
    def _():
        s1, o1 = _bn_coeffs(x_ref[...], g1_ref[...], be1_ref[...])
        c1[0:1, :] = s1
        c1[1:2, :] = o1
        v0[...] = (x_ref[...] * s1 + o1).astype(jnp.bfloat16)

    blk = lap_ref[...].astype(jnp.bfloat16)
    lbf[pl.ds(g * _B, _B), :] = blk
    v1[pl.ds(g * _B, _B), :] = jnp.dot(
        blk, v0[...], preferred_element_type=jnp.float32
    ).astype(jnp.bfloat16)

    @pl.when(g == _G - 1)
    def _():
        w1 = w1_ref[...]
        w2 = w2_ref[...]

        def mmb(a, b):
            return jnp.dot(a, b, preferred_element_type=jnp.float32).astype(
                jnp.bfloat16
            )

        def mmf(a, b):
            return jnp.dot(a, b, preferred_element_type=jnp.float32)

        def rows(ref, i):
            return ref[pl.ds(i * _T, _T), :]

        def sl(i):
            return (pl.ds(i * _T, _T), slice(None))

        for i in range(_NT):
            v2[sl(i)] = 2.0 * mmb(rows(lbf, i), v1[...]) - rows(v0, i)
            hf[sl(i)] = mmf(rows(v0, i), w1[0]) + mmf(rows(v1, i), w1[1])
        hsum = []
        hsq = []
        for i in range(_NT):
            t3_t = 2.0 * mmb(rows(lbf, i), v2[...]) - rows(v1, i)
            acc_t = rows(hf, i) + mmf(rows(v2, i), w1[2]) + mmf(t3_t, w1[3])
            h_t = jnp.maximum(acc_t + b1_ref[...], 0.0)
            hf[sl(i)] = h_t
            hsum.append(jnp.sum(h_t, axis=0, keepdims=True))
            hsq.append(jnp.sum(h_t * h_t, axis=0, keepdims=True))
        mean2 = sum(hsum) * (1.0 / _N)
        var2 = sum(hsq) * (1.0 / _N) - mean2 * mean2
        s2 = g2_ref[...] / jnp.sqrt(var2 + _EPS)
        o2 = be2_ref[...] - mean2 * s2
        for i in range(_NT):
            v0[sl(i)] = (rows(hf, i) * s2 + o2).astype(jnp.bfloat16)
        for i in range(_NT):
            v1[sl(i)] = mmb(rows(lbf, i), v0[...])
            hf[sl(i)] = mmf(rows(v0, i), w2[0])
        for i in range(_NT):
            v2[sl(i)] = 2.0 * mmb(rows(lbf, i), v1[...]) - rows(v0, i)
            hf[sl(i)] = rows(hf, i) + mmf(rows(v1, i), w2[1])
        s1 = c1[0:1, :]
        o1 = c1[1:2, :]
        for i in range(_NT):
            u3_t = 2.0 * mmb(rows(lbf, i), v2[...]) - rows(v1, i)
            acc_t = rows(hf, i) + mmf(rows(v2, i), w2[2]) + mmf(u3_t, w2[3])
            xh_t = rows(x_ref, i) * s1 + o1
            out_ref[sl(i)] = jnp.maximum(xh_t + acc_t + b2_ref[...], 0.0)


def kernel(x, laplacian, bn1_gamma, bn1_beta, W1, b1, bn2_gamma, bn2_beta, W2, b2):
    w1b = W1.astype(jnp.bfloat16)
    w2b = W2.astype(jnp.bfloat16)
    b1r = b1.reshape(1, _C)
    b2r = b2.reshape(1, _C)
    g1r = bn1_gamma.reshape(1, _C)
    be1r = bn1_beta.reshape(1, _C)
    g2r = bn2_gamma.reshape(1, _C)
    be2r = bn2_beta.reshape(1, _C)

    full = lambda shape: pl.BlockSpec(shape, lambda g: tuple(0 for _ in shape))
    return pl.pallas_call(
        _body,
        grid=(_G,),
        in_specs=[
            pl.BlockSpec((_B, _N), lambda g: (g, 0)),
            full((_N, _C)),
            full((4, _C, _C)),
            full((1, _C)),
            full((1, _C)),
            full((1, _C)),
            full((4, _C, _C)),
            full((1, _C)),
            full((1, _C)),
            full((1, _C)),
        ],
        out_specs=full((_N, _C)),
        out_shape=jax.ShapeDtypeStruct((_N, _C), jnp.float32),
        scratch_shapes=[
            pltpu.VMEM((_N, _N), jnp.bfloat16),
            pltpu.VMEM((_N, _C), jnp.bfloat16),
            pltpu.VMEM((_N, _C), jnp.bfloat16),
            pltpu.VMEM((_N, _C), jnp.bfloat16),
            pltpu.VMEM((_N, _C), jnp.float32),
            pltpu.VMEM((2, _C), jnp.float32),
        ],
        compiler_params=pltpu.CompilerParams(
            dimension_semantics=("arbitrary",),
            vmem_limit_bytes=64 * 1024 * 1024,
        ),
    )(laplacian, x, w1b, b1r, g1r, be1r, w2b, b2r, g2r, be2r)


# speedup vs baseline: 1.0458x; 1.0275x over previous
import jax
import jax.numpy as jnp
from jax.experimental import pallas as pl
from jax.experimental.pallas import tpu as pltpu

_N = 4096
_C = 256
_EPS = 1e-5
_B = 256
_G = _N // _B
_T = 1024
_NT = _N // _T


def _bn_coeffs(v, gamma, beta):
    mean = jnp.mean(v, axis=0, keepdims=True)
    var = jnp.mean((v - mean) ** 2, axis=0, keepdims=True)
    rstd = gamma / jnp.sqrt(var + _EPS)
    return rstd, beta - mean * rstd


def _body(lap_ref, x_ref, w1_ref, b1_ref, g1_ref, be1_ref,
          w2_ref, b2_ref, g2_ref, be2_ref,
          out_ref, lbf, v0, v1, v2, hf, c1):
    g = pl.program_id(0)

    @pl.when(g == 0)
    def _():
        s1, o1 = _bn_coeffs(x_ref[...], g1_ref[...], be1_ref[...])
        c1[0:1, :] = s1
        c1[1:2, :] = o1
        v0[...] = (x_ref[...] * s1 + o1).astype(jnp.bfloat16)

    blk = lap_ref[...].astype(jnp.bfloat16)
    lbf[pl.ds(g * _B, _B), :] = blk
    t1_blk = jnp.dot(
        blk, v0[...], preferred_element_type=jnp.float32
    ).astype(jnp.bfloat16)
    v1[pl.ds(g * _B, _B), :] = t1_blk
    hf[pl.ds(g * _B, _B), :] = jnp.dot(
        v0[pl.ds(g * _B, _B), :], w1_ref[0], preferred_element_type=jnp.float32
    ) + jnp.dot(t1_blk, w1_ref[1], preferred_element_type=jnp.float32)

    @pl.when(g == _G - 1)
    def _():
        w1 = w1_ref[...]
        w2 = w2_ref[...]

        def mmb(a, b):
            return jnp.dot(a, b, preferred_element_type=jnp.float32).astype(
                jnp.bfloat16
            )

        def mmf(a, b):
            return jnp.dot(a, b, preferred_element_type=jnp.float32)

        def rows(ref, i):
            return ref[pl.ds(i * _T, _T), :]

        def sl(i):
            return (pl.ds(i * _T, _T), slice(None))

        for i in range(_NT):
            v2[sl(i)] = 2.0 * mmb(rows(lbf, i), v1[...]) - rows(v0, i)
        hsum = []
        hsq = []
        for i in range(_NT):
            t3_t = 2.0 * mmb(rows(lbf, i), v2[...]) - rows(v1, i)
            acc_t = rows(hf, i) + mmf(rows(v2, i), w1[2]) + mmf(t3_t, w1[3])
            h_t = jnp.maximum(acc_t + b1_ref[...], 0.0)
            hf[sl(i)] = h_t
            hsum.append(jnp.sum(h_t, axis=0, keepdims=True))
            hsq.append(jnp.sum(h_t * h_t, axis=0, keepdims=True))
        mean2 = sum(hsum) * (1.0 / _N)
        var2 = sum(hsq) * (1.0 / _N) - mean2 * mean2
        s2 = g2_ref[...] / jnp.sqrt(var2 + _EPS)
        o2 = be2_ref[...] - mean2 * s2
        for i in range(_NT):
            v0[sl(i)] = (rows(hf, i) * s2 + o2).astype(jnp.bfloat16)
        for i in range(_NT):
            v1[sl(i)] = mmb(rows(lbf, i), v0[...])
            hf[sl(i)] = mmf(rows(v0, i), w2[0])
        for i in range(_NT):
            v2[sl(i)] = 2.0 * mmb(rows(lbf, i), v1[...]) - rows(v0, i)
            hf[sl(i)] = rows(hf, i) + mmf(rows(v1, i), w2[1])
        s1 = c1[0:1, :]
        o1 = c1[1:2, :]
        for i in range(_NT):
            u3_t = 2.0 * mmb(rows(lbf, i), v2[...]) - rows(v1, i)
            acc_t = rows(hf, i) + mmf(rows(v2, i), w2[2]) + mmf(u3_t, w2[3])
            xh_t = rows(x_ref, i) * s1 + o1
            out_ref[sl(i)] = jnp.maximum(xh_t + acc_t + b2_ref[...], 0.0)


def kernel(x, laplacian, bn1_gamma, bn1_beta, W1, b1, bn2_gamma, bn2_beta, W2, b2):
    w1b = W1.astype(jnp.bfloat16)
    w2b = W2.astype(jnp.bfloat16)
    b1r = b1.reshape(1, _C)
    b2r = b2.reshape(1, _C)
    g1r = bn1_gamma.reshape(1, _C)
    be1r = bn1_beta.reshape(1, _C)
    g2r = bn2_gamma.reshape(1, _C)
    be2r = bn2_beta.reshape(1, _C)

    full = lambda shape: pl.BlockSpec(shape, lambda g: tuple(0 for _ in shape))
    return pl.pallas_call(
        _body,
        grid=(_G,),
        in_specs=[
            pl.BlockSpec((_B, _N), lambda g: (g, 0)),
            full((_N, _C)),
            full((4, _C, _C)),
            full((1, _C)),
            full((1, _C)),
            full((1, _C)),
            full((4, _C, _C)),
            full((1, _C)),
            full((1, _C)),
            full((1, _C)),
        ],
        out_specs=full((_N, _C)),
        out_shape=jax.ShapeDtypeStruct((_N, _C), jnp.float32),
        scratch_shapes=[
            pltpu.VMEM((_N, _N), jnp.bfloat16),
            pltpu.VMEM((_N, _C), jnp.bfloat16),
            pltpu.VMEM((_N, _C), jnp.bfloat16),
            pltpu.VMEM((_N, _C), jnp.bfloat16),
            pltpu.VMEM((_N, _C), jnp.float32),
            pltpu.VMEM((2, _C), jnp.float32),
        ],
        compiler_params=pltpu.CompilerParams(
            dimension_semantics=("arbitrary",),
            vmem_limit_bytes=64 * 1024 * 1024,
        ),
    )(laplacian, x, w1b, b1r, g1r, be1r, w2b, b2r, g2r, be2r)
